# baseline (speedup 1.0000x reference)
import functools

import jax
import jax.numpy as jnp
from jax import lax
from jax.experimental import pallas as pl
from jax.experimental.pallas import tpu as pltpu
from jax.experimental.pallas import tpu_sc as plsc

N_KEYS = 1000
DEFAULT = -1.0

L = 16
NC, NS = 2, 16
NW = NC * NS
B_TOTAL = 16384 * 26
BPW = B_TOTAL // NW
NCHUNK = 8
CB = BPW // NCHUNK

KPAD = 1008
TBL = 1024

_mesh = plsc.VectorSubcoreMesh(
    core_axis_name="c", subcore_axis_name="s", num_cores=NC, num_subcores=NS
)


@functools.partial(
    pl.kernel,
    out_type=jax.ShapeDtypeStruct((B_TOTAL,), jnp.float32),
    mesh=_mesh,
    scratch_types=dict(
        keys_v=pltpu.VMEM((KPAD,), jnp.int32),
        vals_v=pltpu.VMEM((KPAD,), jnp.float32),
        table_v=pltpu.VMEM((TBL,), jnp.float32),
        idx_v=pltpu.VMEM((BPW,), jnp.int32),
        res_v=pltpu.VMEM((BPW,), jnp.float32),
        in_sem0=pltpu.SemaphoreType.DMA,
        in_sem1=pltpu.SemaphoreType.DMA,
        in_sem2=pltpu.SemaphoreType.DMA,
        in_sem3=pltpu.SemaphoreType.DMA,
        in_sem4=pltpu.SemaphoreType.DMA,
        in_sem5=pltpu.SemaphoreType.DMA,
        in_sem6=pltpu.SemaphoreType.DMA,
        in_sem7=pltpu.SemaphoreType.DMA,
        key_sem=pltpu.SemaphoreType.DMA,
        val_sem=pltpu.SemaphoreType.DMA,
        out_sem=pltpu.SemaphoreType.DMA,
    ),
    compiler_params=pltpu.CompilerParams(
        needs_layout_passes=False, skip_device_barrier=True
    ),
)
def _lookup_kernel(ids_hbm, keys_hbm, vals_hbm, out_hbm,
                   keys_v, vals_v, table_v, idx_v, res_v,
                   in_sem0, in_sem1, in_sem2, in_sem3,
                   in_sem4, in_sem5, in_sem6, in_sem7,
                   key_sem, val_sem, out_sem):
    wid = lax.axis_index("s") * NC + lax.axis_index("c")
    base = wid * BPW
    in_sems = (in_sem0, in_sem1, in_sem2, in_sem3,
               in_sem4, in_sem5, in_sem6, in_sem7)

    in_cps = [
        pltpu.async_copy(
            ids_hbm.at[pl.ds(base + k * CB, CB)],
            idx_v.at[pl.ds(k * CB, CB)],
            in_sems[k],
        )
        for k in range(NCHUNK)
    ]
    key_cp = pltpu.async_copy(keys_hbm, keys_v.at[pl.ds(0, N_KEYS)], key_sem)
    val_cp = pltpu.async_copy(vals_hbm, vals_v.at[pl.ds(0, N_KEYS)], val_sem)

    default = jnp.full((L,), DEFAULT, dtype=jnp.float32)

    @plsc.parallel_loop(0, TBL, step=L)
    def _fill(i):
        table_v[pl.ds(i, L)] = default

    key_cp.wait()
    val_cp.wait()

    lane = lax.iota(jnp.int32, L)
    tail_mask = lane < jnp.full((L,), N_KEYS % L, dtype=jnp.int32)

    @plsc.parallel_loop(0, KPAD, step=L)
    def _scatter(i):
        k = keys_v[pl.ds(i, L)]
        v = vals_v[pl.ds(i, L)]
        ones = jnp.full((L,), 1, dtype=jnp.int32)
        is_full = jnp.where(i + L <= N_KEYS, ones, jnp.zeros_like(ones))
        m = (is_full > 0) | tail_mask
        plsc.store_scatter(table_v, [k], v, mask=m)

    zero = jnp.zeros((L,), dtype=jnp.int32)
    top = jnp.full((L,), N_KEYS - 1, dtype=jnp.int32)

    out_cps = []
    for k in range(NCHUNK):
        in_cps[k].wait()

        @plsc.parallel_loop(k * CB, (k + 1) * CB, step=L, unroll=16)
        def _gather(i):
            idx = idx_v[pl.ds(i, L)]
            in_range = (idx >= zero) & (idx <= top)
            safe = jnp.minimum(jnp.maximum(idx, zero), top)
            g = plsc.load_gather(table_v, [safe])
            res_v[pl.ds(i, L)] = jnp.where(in_range, g, default)

        out_cps.append(
            pltpu.async_copy(
                res_v.at[pl.ds(k * CB, CB)],
                out_hbm.at[pl.ds(base + k * CB, CB)],
                out_sem,
            )
        )

    for cp in out_cps:
        cp.wait()


def kernel(inputs, keys, values):
    out = _lookup_kernel(inputs.reshape(-1), keys, values)
    return out.reshape(inputs.shape)


# speedup vs baseline: 1.0270x; 1.0270x over previous
import functools

import jax
import jax.numpy as jnp
from jax import lax
from jax.experimental import pallas as pl
from jax.experimental.pallas import tpu as pltpu
from jax.experimental.pallas import tpu_sc as plsc

N_KEYS = 1000
DEFAULT = -1.0

L = 16
NC, NS = 2, 16
NW = NC * NS
B_TOTAL = 16384 * 26
BPW = B_TOTAL // NW
NCHUNK = 2
CB = BPW // NCHUNK

KPAD = 1008
TBL = 1024

_mesh = plsc.VectorSubcoreMesh(
    core_axis_name="c", subcore_axis_name="s", num_cores=NC, num_subcores=NS
)


@functools.partial(
    pl.kernel,
    out_type=jax.ShapeDtypeStruct((B_TOTAL,), jnp.float32),
    mesh=_mesh,
    scratch_types=dict(
        keys_v=pltpu.VMEM((KPAD,), jnp.int32),
        vals_v=pltpu.VMEM((KPAD,), jnp.float32),
        table_v=pltpu.VMEM((TBL,), jnp.float32),
        idx_v=pltpu.VMEM((BPW,), jnp.int32),
        res_v=pltpu.VMEM((BPW,), jnp.float32),
        in_sem0=pltpu.SemaphoreType.DMA,
        in_sem1=pltpu.SemaphoreType.DMA,
        in_sem2=pltpu.SemaphoreType.DMA,
        in_sem3=pltpu.SemaphoreType.DMA,
        key_sem=pltpu.SemaphoreType.DMA,
        val_sem=pltpu.SemaphoreType.DMA,
        out_sem=pltpu.SemaphoreType.DMA,
    ),
    compiler_params=pltpu.CompilerParams(
        needs_layout_passes=False, skip_device_barrier=True
    ),
)
def _lookup_kernel(ids_hbm, keys_hbm, vals_hbm, out_hbm,
                   keys_v, vals_v, table_v, idx_v, res_v,
                   in_sem0, in_sem1, in_sem2, in_sem3,
                   key_sem, val_sem, out_sem):
    wid = lax.axis_index("s") * NC + lax.axis_index("c")
    base = wid * BPW
    in_sems = (in_sem0, in_sem1, in_sem2, in_sem3)[:NCHUNK]

    in_cps = [
        pltpu.async_copy(
            ids_hbm.at[pl.ds(base + k * CB, CB)],
            idx_v.at[pl.ds(k * CB, CB)],
            in_sems[k],
        )
        for k in range(NCHUNK)
    ]
    key_cp = pltpu.async_copy(keys_hbm, keys_v.at[pl.ds(0, N_KEYS)], key_sem)
    val_cp = pltpu.async_copy(vals_hbm, vals_v.at[pl.ds(0, N_KEYS)], val_sem)

    default = jnp.full((L,), DEFAULT, dtype=jnp.float32)

    @plsc.parallel_loop(0, TBL, step=L)
    def _fill(i):
        table_v[pl.ds(i, L)] = default

    key_cp.wait()
    val_cp.wait()

    n_full = N_KEYS - N_KEYS % L

    @plsc.parallel_loop(0, n_full, step=L)
    def _scatter(i):
        plsc.store_scatter(table_v, [keys_v[pl.ds(i, L)]], vals_v[pl.ds(i, L)])

    lane = lax.iota(jnp.int32, L)
    tail_mask = lane < jnp.full((L,), N_KEYS % L, dtype=jnp.int32)
    plsc.store_scatter(
        table_v, [keys_v[pl.ds(n_full, L)]], vals_v[pl.ds(n_full, L)],
        mask=tail_mask,
    )

    zero = jnp.zeros((L,), dtype=jnp.int32)
    top = jnp.full((L,), N_KEYS - 1, dtype=jnp.int32)

    out_cps = []
    for k in range(NCHUNK):
        in_cps[k].wait()

        @plsc.parallel_loop(k * CB, (k + 1) * CB, step=L, unroll=8)
        def _gather(i):
            idx = idx_v[pl.ds(i, L)]
            in_range = (idx >= zero) & (idx <= top)
            safe = jnp.minimum(jnp.maximum(idx, zero), top)
            g = plsc.load_gather(table_v, [safe])
            res_v[pl.ds(i, L)] = jnp.where(in_range, g, default)

        out_cps.append(
            pltpu.async_copy(
                res_v.at[pl.ds(k * CB, CB)],
                out_hbm.at[pl.ds(base + k * CB, CB)],
                out_sem,
            )
        )

    for cp in out_cps:
        cp.wait()


def kernel(inputs, keys, values):
    out = _lookup_kernel(inputs.reshape(-1), keys, values)
    return out.reshape(inputs.shape)
